# Initial kernel scaffold; baseline (speedup 1.0000x reference)
import jax
import jax.numpy as jnp
from jax.experimental import pallas as pl


def kernel(user_emb, item_emb, ui_edge_index, ui_values, ii_edge_index, ii_values):
    raise NotImplementedError("write your pallas kernel here")


# speedup vs baseline: 22.9843x; 22.9843x over previous
import functools

import jax
import jax.numpy as jnp
from jax import lax
from jax.experimental import pallas as pl
from jax.experimental.pallas import tpu as pltpu
from jax.experimental.pallas import tpu_sc as plsc

_DIM = 16
_SUB = 125
_CROWS = 5
_NCORES = 2
_NSUB = 16
_NW = _NCORES * _NSUB


@functools.lru_cache(maxsize=None)
def _make_spmm(n_rows, n_edges):
    rows_total = n_edges // _SUB
    rows_per_worker = rows_total // _NW
    chunks = rows_per_worker // _CROWS
    out_rows_per_tile = n_rows // _NSUB
    mesh = plsc.VectorSubcoreMesh(core_axis_name="c", subcore_axis_name="s")

    @functools.partial(
        pl.kernel,
        out_type=(
            jax.ShapeDtypeStruct((n_rows, _DIM), jnp.float32),
            jax.ShapeDtypeStruct((n_rows, _DIM), jnp.float32),
        ),
        mesh=mesh,
        compiler_params=pltpu.CompilerParams(use_tc_tiling_on_sc=False,
                                             needs_layout_passes=False),
        scratch_types=[
            pltpu.VMEM((_CROWS, _SUB), jnp.int32),
            pltpu.VMEM((_CROWS, _SUB), jnp.int32),
            pltpu.VMEM((_CROWS, _SUB), jnp.float32),
            pltpu.VMEM((_CROWS, _SUB, _DIM), jnp.float32),
            pltpu.VMEM((_CROWS, _SUB, _DIM), jnp.float32),
            pltpu.VMEM_SHARED((n_rows, _DIM), jnp.float32),
            pltpu.SemaphoreType.DMA,
            pltpu.SemaphoreType.DMA,
        ],
    )
    def spmm(x_hbm, src_hbm, dst_hbm, val_hbm, zero_hbm, out0_hbm, out1_hbm,
             src_v, dst_v, val_v, rows_v, scaled_v, acc, gsem, ssem):
        cid = lax.axis_index("c")
        sid = lax.axis_index("s")
        wid = sid * _NCORES + cid
        r0 = sid * out_rows_per_tile
        pltpu.sync_copy(zero_hbm.at[pl.ds(r0, out_rows_per_tile)],
                        acc.at[pl.ds(r0, out_rows_per_tile)])
        plsc.subcore_barrier()

        row_base = wid * rows_per_worker

        def chunk_body(k, carry):
            base = row_base + k * _CROWS
            pltpu.sync_copy(src_hbm.at[pl.ds(base, _CROWS)], src_v)
            pltpu.sync_copy(dst_hbm.at[pl.ds(base, _CROWS)], dst_v)
            pltpu.sync_copy(val_hbm.at[pl.ds(base, _CROWS)], val_v)
            gds = [pltpu.async_copy(x_hbm.at[src_v.at[j]], rows_v.at[j], gsem)
                   for j in range(_CROWS)]
            for d in gds:
                d.wait()
            for j in range(_CROWS):
                @plsc.parallel_loop(0, _SUB, unroll=5)
                def _scale(i, _j=j):
                    jj = jnp.full((_DIM,), _j, jnp.int32)
                    ii = jnp.full((_DIM,), i, jnp.int32)
                    w = plsc.load_gather(val_v, [jj, ii])
                    scaled_v[_j, i, :] = rows_v[_j, i, :] * w
            sds = [pltpu.async_copy(scaled_v.at[j], acc.at[dst_v.at[j]], ssem,
                                    add=True)
                   for j in range(_CROWS)]
            for d in sds:
                d.wait()
            return carry

        lax.fori_loop(0, chunks, chunk_body, 0)
        plsc.subcore_barrier()
        out = [out0_hbm, out1_hbm]
        for c in range(_NCORES):
            @pl.when(cid == c)
            def _():
                pltpu.sync_copy(acc.at[pl.ds(r0, out_rows_per_tile)],
                                out[c].at[pl.ds(r0, out_rows_per_tile)])

    return spmm


_EW_BLOCK = 4096


@functools.lru_cache(maxsize=None)
def _make_ew(n_in, rows, op):
    grid = pl.cdiv(rows, _EW_BLOCK)
    spec = pl.BlockSpec((_EW_BLOCK, 128), lambda i: (i, 0))

    def body(*refs):
        ins = [r[...] for r in refs[:-1]]
        refs[-1][...] = op(ins)

    return pl.pallas_call(
        body,
        grid=(grid,),
        in_specs=[spec] * n_in,
        out_specs=spec,
        out_shape=jax.ShapeDtypeStruct((rows, 128), jnp.float32),
    )


def _ew(op, *arrays):
    rows = arrays[0].size // 128
    ins = [a.reshape(rows, 128) for a in arrays]
    out = _make_ew(len(ins), rows, op)(*ins)
    return out.reshape(arrays[0].shape)


_ADD2 = lambda xs: xs[0] + xs[1]
_MEAN4 = lambda xs: (xs[0] + xs[1] + xs[2] + xs[3]) * 0.25
_MEAN3 = lambda xs: (xs[0] + xs[1] + xs[2]) * (1.0 / 3.0)
_LERP = lambda xs: (xs[0] + xs[1]) * 0.5


def _spmm(x, src, dst, val, zeros, n_rows, n_edges):
    p0, p1 = _make_spmm(n_rows, n_edges)(x, src, dst, val, zeros)
    return _ew(_ADD2, p0, p1)


def kernel(user_emb, item_emb, ui_edge_index, ui_values, ii_edge_index, ii_values):
    n_ui = user_emb.shape[0] + item_emb.shape[0]
    n_ii = item_emb.shape[0]
    e_ui = ui_values.shape[0]
    e_ii = ii_values.shape[0]

    ui_src = ui_edge_index[1].reshape(e_ui // _SUB, _SUB)
    ui_dst = ui_edge_index[0].reshape(e_ui // _SUB, _SUB)
    ui_val = ui_values.reshape(e_ui // _SUB, _SUB)
    ii_src = ii_edge_index[1].reshape(e_ii // _SUB, _SUB)
    ii_dst = ii_edge_index[0].reshape(e_ii // _SUB, _SUB)
    ii_val = ii_values.reshape(e_ii // _SUB, _SUB)

    zeros_ui = jnp.zeros((n_ui, _DIM), jnp.float32)
    zeros_ii = jnp.zeros((n_ii, _DIM), jnp.float32)

    x = jnp.concatenate([user_emb, item_emb], axis=0)
    xs = [x]
    for _ in range(3):
        x = _spmm(x, ui_src, ui_dst, ui_val, zeros_ui, n_ui, e_ui)
        xs.append(x)
    light_out_ui = _ew(_MEAN4, *xs)
    users_emb_ui = light_out_ui[: user_emb.shape[0]]
    items_emb_ui = light_out_ui[user_emb.shape[0]:]

    y = item_emb
    ys = [y]
    for _ in range(2):
        y = _spmm(y, ii_src, ii_dst, ii_val, zeros_ii, n_ii, e_ii)
        ys.append(y)
    items_emb_semantic = _ew(_MEAN3, *ys)

    items_fused = _ew(_LERP, items_emb_ui, items_emb_semantic)
    return (users_emb_ui, items_emb_ui, items_emb_semantic, items_fused)


# speedup vs baseline: 25.3613x; 1.1034x over previous
import functools

import jax
import jax.numpy as jnp
from jax import lax
from jax.experimental import pallas as pl
from jax.experimental.pallas import tpu as pltpu
from jax.experimental.pallas import tpu_sc as plsc

_DIM = 16
_SUB = 125
_NCORES = 2
_NSUB = 16
_NW = _NCORES * _NSUB


@functools.lru_cache(maxsize=None)
def _make_spmm(n_rows, n_edges, crows):
    rows_total = n_edges // _SUB
    rows_per_worker = rows_total // _NW
    chunks = rows_per_worker // crows
    out_rows_per_tile = n_rows // _NSUB
    mesh = plsc.VectorSubcoreMesh(core_axis_name="c", subcore_axis_name="s")

    @functools.partial(
        pl.kernel,
        out_type=jax.ShapeDtypeStruct((_NCORES, n_rows, _DIM), jnp.float32),
        mesh=mesh,
        compiler_params=pltpu.CompilerParams(use_tc_tiling_on_sc=False,
                                             needs_layout_passes=False),
        scratch_types=[
            pltpu.VMEM((crows, _SUB), jnp.int32),
            pltpu.VMEM((crows, _SUB), jnp.int32),
            pltpu.VMEM((crows, _SUB), jnp.float32),
            pltpu.VMEM((crows, _SUB, _DIM), jnp.float32),
            pltpu.VMEM_SHARED((n_rows, _DIM), jnp.float32),
            pltpu.SemaphoreType.DMA,
            pltpu.SemaphoreType.DMA,
            pltpu.SemaphoreType.DMA,
        ],
    )
    def spmm(x_hbm, src_hbm, dst_hbm, val_hbm, zero_hbm, out_hbm,
             src_v, dst_v, val_v, rows_v, acc, isem, gsem, ssem):
        cid = lax.axis_index("c")
        sid = lax.axis_index("s")
        wid = sid * _NCORES + cid
        r0 = sid * out_rows_per_tile
        pltpu.sync_copy(zero_hbm.at[pl.ds(r0, out_rows_per_tile)],
                        acc.at[pl.ds(r0, out_rows_per_tile)])
        plsc.subcore_barrier()

        row_base = wid * rows_per_worker

        def chunk_body(k, carry):
            base = row_base + k * crows
            ids = [pltpu.async_copy(src_hbm.at[pl.ds(base, crows)], src_v, isem),
                   pltpu.async_copy(dst_hbm.at[pl.ds(base, crows)], dst_v, isem),
                   pltpu.async_copy(val_hbm.at[pl.ds(base, crows)], val_v, isem)]
            for d in ids:
                d.wait()
            gds = []
            for j in range(crows):
                gds.append(
                    pltpu.async_copy(x_hbm.at[src_v.at[j]], rows_v.at[j], gsem))
                if j >= 4:
                    gds[j - 4].wait()
            for d in gds[max(0, crows - 4):]:
                d.wait()
            for j in range(crows):
                @plsc.parallel_loop(0, _SUB, unroll=5)
                def _scale(i, _j=j):
                    jj = jnp.full((_DIM,), _j, jnp.int32)
                    ii = jnp.full((_DIM,), i, jnp.int32)
                    w = plsc.load_gather(val_v, [jj, ii])
                    rows_v[_j, i, :] = rows_v[_j, i, :] * w
            sds = []
            for j in range(crows):
                sds.append(
                    pltpu.async_copy(rows_v.at[j], acc.at[dst_v.at[j]], ssem,
                                     add=True))
                if j >= 4:
                    sds[j - 4].wait()
            for d in sds[max(0, crows - 4):]:
                d.wait()
            return carry

        lax.fori_loop(0, chunks, chunk_body, 0)
        plsc.subcore_barrier()
        pltpu.sync_copy(acc.at[pl.ds(r0, out_rows_per_tile)],
                        out_hbm.at[cid].at[pl.ds(r0, out_rows_per_tile)])

    return spmm


_EW_BLOCK = 4096


@functools.lru_cache(maxsize=None)
def _make_ew(n_in, rows, op, n_out=1):
    grid = pl.cdiv(rows, _EW_BLOCK)
    spec = pl.BlockSpec((_EW_BLOCK, 128), lambda i: (i, 0))

    def body(*refs):
        ins = [r[...] for r in refs[:n_in]]
        outs = op(ins)
        if n_out == 1:
            refs[-1][...] = outs
        else:
            for o_ref, o in zip(refs[n_in:], outs):
                o_ref[...] = o

    shape = jax.ShapeDtypeStruct((rows, 128), jnp.float32)
    return pl.pallas_call(
        body,
        grid=(grid,),
        in_specs=[spec] * n_in,
        out_specs=spec if n_out == 1 else [spec] * n_out,
        out_shape=shape if n_out == 1 else [shape] * n_out,
    )


def _ew(op, *arrays, n_out=1):
    rows = arrays[0].size // 128
    ins = [a.reshape(rows, 128) for a in arrays]
    out = _make_ew(len(ins), rows, op, n_out)(*ins)
    if n_out == 1:
        return out.reshape(arrays[0].shape)
    return [o.reshape(arrays[0].shape) for o in out]


_ADD2 = lambda xs: xs[0] + xs[1]
_MEAN4P = lambda xs: (xs[0] + xs[1] + xs[2] + xs[3] + xs[4]) * 0.25
_SEM_FUSE = lambda xs: (
    (xs[0] + xs[1] + xs[2] + xs[3]) * (1.0 / 3.0),
    ((xs[0] + xs[1] + xs[2] + xs[3]) * (1.0 / 3.0) + xs[4]) * 0.5,
)


def _spmm_partials(x, src, dst, val, zeros, n_rows, n_edges, crows):
    parts = _make_spmm(n_rows, n_edges, crows)(x, src, dst, val, zeros)
    return parts[0], parts[1]


def kernel(user_emb, item_emb, ui_edge_index, ui_values, ii_edge_index, ii_values):
    n_users = user_emb.shape[0]
    n_ui = n_users + item_emb.shape[0]
    n_ii = item_emb.shape[0]
    e_ui = ui_values.shape[0]
    e_ii = ii_values.shape[0]

    ui_src = ui_edge_index[1].reshape(e_ui // _SUB, _SUB)
    ui_dst = ui_edge_index[0].reshape(e_ui // _SUB, _SUB)
    ui_val = ui_values.reshape(e_ui // _SUB, _SUB)
    ii_src = ii_edge_index[1].reshape(e_ii // _SUB, _SUB)
    ii_dst = ii_edge_index[0].reshape(e_ii // _SUB, _SUB)
    ii_val = ii_values.reshape(e_ii // _SUB, _SUB)

    zeros_ui = jnp.zeros((n_ui, _DIM), jnp.float32)
    zeros_ii = jnp.zeros((n_ii, _DIM), jnp.float32)

    x0 = jnp.concatenate([user_emb, item_emb], axis=0)
    p0, p1 = _spmm_partials(x0, ui_src, ui_dst, ui_val, zeros_ui, n_ui, e_ui, 10)
    x1 = _ew(_ADD2, p0, p1)
    p0, p1 = _spmm_partials(x1, ui_src, ui_dst, ui_val, zeros_ui, n_ui, e_ui, 10)
    x2 = _ew(_ADD2, p0, p1)
    p0, p1 = _spmm_partials(x2, ui_src, ui_dst, ui_val, zeros_ui, n_ui, e_ui, 10)
    light_out_ui = _ew(_MEAN4P, x0, x1, x2, p0, p1)
    users_emb_ui = light_out_ui[:n_users]
    items_emb_ui = light_out_ui[n_users:]

    q0, q1 = _spmm_partials(item_emb, ii_src, ii_dst, ii_val, zeros_ii,
                            n_ii, e_ii, 10)
    y1 = _ew(_ADD2, q0, q1)
    q0, q1 = _spmm_partials(y1, ii_src, ii_dst, ii_val, zeros_ii,
                            n_ii, e_ii, 10)
    items_emb_semantic, items_fused = _ew(
        _SEM_FUSE, item_emb, y1, q0, q1, items_emb_ui, n_out=2)

    return (users_emb_ui, items_emb_ui, items_emb_semantic, items_fused)


# speedup vs baseline: 31.1035x; 1.2264x over previous
import functools

import jax
import jax.numpy as jnp
from jax import lax
from jax.experimental import pallas as pl
from jax.experimental.pallas import tpu as pltpu
from jax.experimental.pallas import tpu_sc as plsc

_DIM = 16
_SUB = 125
_NCORES = 2
_NSUB = 16
_NW = _NCORES * _NSUB


@functools.lru_cache(maxsize=None)
def _make_spmm(n_rows, n_edges, crows):
    rows_total = n_edges // _SUB
    rows_per_worker = rows_total // _NW
    chunks = rows_per_worker // crows
    assert chunks >= 4 and chunks % 2 == 0
    pairs = (chunks - 2) // 2
    out_rows_per_tile = n_rows // _NSUB
    mesh = plsc.VectorSubcoreMesh(core_axis_name="c", subcore_axis_name="s")

    idx_i = pltpu.VMEM((crows, _SUB), jnp.int32)
    idx_f = pltpu.VMEM((crows, _SUB), jnp.float32)
    rows_t = pltpu.VMEM((crows, _SUB, _DIM), jnp.float32)
    sem = pltpu.SemaphoreType.DMA

    @functools.partial(
        pl.kernel,
        out_type=jax.ShapeDtypeStruct((_NCORES, n_rows, _DIM), jnp.float32),
        mesh=mesh,
        compiler_params=pltpu.CompilerParams(use_tc_tiling_on_sc=False,
                                             needs_layout_passes=False),
        scratch_types=[
            idx_i, idx_i, idx_f, rows_t,
            idx_i, idx_i, idx_f, rows_t,
            pltpu.VMEM_SHARED((n_rows, _DIM), jnp.float32),
            sem, sem, sem, sem, sem, sem,
        ],
    )
    def spmm(x_hbm, src_hbm, dst_hbm, val_hbm, zero_hbm, out_hbm,
             src0, dst0, val0, rows0, src1, dst1, val1, rows1, acc,
             isem0, gsem0, ssem0, isem1, gsem1, ssem1):
        cid = lax.axis_index("c")
        sid = lax.axis_index("s")
        wid = sid * _NCORES + cid
        r0 = sid * out_rows_per_tile
        pltpu.sync_copy(zero_hbm.at[pl.ds(r0, out_rows_per_tile)],
                        acc.at[pl.ds(r0, out_rows_per_tile)])
        plsc.subcore_barrier()

        row_base = wid * rows_per_worker
        B = [
            dict(src=src0, dst=dst0, val=val0, rows=rows0,
                 isem=isem0, gsem=gsem0, ssem=ssem0),
            dict(src=src1, dst=dst1, val=val1, rows=rows1,
                 isem=isem1, gsem=gsem1, ssem=ssem1),
        ]

        def fire_idx(k, b):
            base = row_base + k * crows
            pltpu.async_copy(src_hbm.at[pl.ds(base, crows)], b["src"], b["isem"])
            pltpu.async_copy(dst_hbm.at[pl.ds(base, crows)], b["dst"], b["isem"])
            pltpu.async_copy(val_hbm.at[pl.ds(base, crows)], b["val"], b["isem"])

        def wait_idx(b):
            for ref in (b["src"], b["dst"], b["val"]):
                pltpu.make_async_copy(
                    src_hbm.at[pl.ds(0, crows)], ref, b["isem"]).wait()

        def fire_gather(b):
            for j in range(crows):
                pltpu.async_copy(x_hbm.at[b["src"].at[j]], b["rows"].at[j],
                                 b["gsem"])

        def wait_gather(b):
            for j in range(crows):
                pltpu.make_async_copy(x_hbm.at[b["src"].at[j]],
                                      b["rows"].at[j], b["gsem"]).wait()

        def compute(b):
            val_v, rows_v = b["val"], b["rows"]
            for j in range(crows):
                @plsc.parallel_loop(0, _SUB, unroll=5)
                def _scale(i, _j=j):
                    jj = jnp.full((_DIM,), _j, jnp.int32)
                    ii = jnp.full((_DIM,), i, jnp.int32)
                    w = plsc.load_gather(val_v, [jj, ii])
                    rows_v[_j, i, :] = rows_v[_j, i, :] * w

        def fire_scatter(b):
            for j in range(crows):
                pltpu.async_copy(b["rows"].at[j], acc.at[b["dst"].at[j]],
                                 b["ssem"], add=True)

        def wait_scatter(b):
            for j in range(crows):
                pltpu.make_async_copy(b["rows"].at[j], acc.at[b["dst"].at[j]],
                                      b["ssem"]).wait()

        def body(k, b, b1, fire_next_gather=True, fire_next_idx=True):
            if fire_next_gather:
                wait_idx(b1)
                fire_gather(b1)
            wait_gather(b)
            compute(b)
            fire_scatter(b)
            wait_scatter(b)
            if fire_next_idx:
                fire_idx(k + 2, b)

        fire_idx(0, B[0])
        fire_idx(1, B[1])
        wait_idx(B[0])
        fire_gather(B[0])

        def pair_body(q, carry):
            k = 2 * q
            body(k, B[0], B[1])
            body(k + 1, B[1], B[0])
            return carry

        lax.fori_loop(0, pairs, pair_body, 0)

        body(chunks - 2, B[0], B[1], fire_next_idx=False)
        body(chunks - 1, B[1], B[0], fire_next_gather=False,
             fire_next_idx=False)

        plsc.subcore_barrier()
        pltpu.sync_copy(acc.at[pl.ds(r0, out_rows_per_tile)],
                        out_hbm.at[cid].at[pl.ds(r0, out_rows_per_tile)])

    return spmm


_EW_BLOCK = 4096


@functools.lru_cache(maxsize=None)
def _make_ew(n_in, rows, op, n_out=1):
    grid = pl.cdiv(rows, _EW_BLOCK)
    spec = pl.BlockSpec((_EW_BLOCK, 128), lambda i: (i, 0))

    def body(*refs):
        ins = [r[...] for r in refs[:n_in]]
        outs = op(ins)
        if n_out == 1:
            refs[-1][...] = outs
        else:
            for o_ref, o in zip(refs[n_in:], outs):
                o_ref[...] = o

    shape = jax.ShapeDtypeStruct((rows, 128), jnp.float32)
    return pl.pallas_call(
        body,
        grid=(grid,),
        in_specs=[spec] * n_in,
        out_specs=spec if n_out == 1 else [spec] * n_out,
        out_shape=shape if n_out == 1 else [shape] * n_out,
    )


def _ew(op, *arrays, n_out=1):
    rows = arrays[0].size // 128
    ins = [a.reshape(rows, 128) for a in arrays]
    out = _make_ew(len(ins), rows, op, n_out)(*ins)
    if n_out == 1:
        return out.reshape(arrays[0].shape)
    return [o.reshape(arrays[0].shape) for o in out]


_ADD2 = lambda xs: xs[0] + xs[1]
_MEAN4P = lambda xs: (xs[0] + xs[1] + xs[2] + xs[3] + xs[4]) * 0.25
_SEM_FUSE = lambda xs: (
    (xs[0] + xs[1] + xs[2] + xs[3]) * (1.0 / 3.0),
    ((xs[0] + xs[1] + xs[2] + xs[3]) * (1.0 / 3.0) + xs[4]) * 0.5,
)


def _spmm_partials(x, src, dst, val, zeros, n_rows, n_edges, crows):
    parts = _make_spmm(n_rows, n_edges, crows)(x, src, dst, val, zeros)
    return parts[0], parts[1]


def kernel(user_emb, item_emb, ui_edge_index, ui_values, ii_edge_index, ii_values):
    n_users = user_emb.shape[0]
    n_ui = n_users + item_emb.shape[0]
    n_ii = item_emb.shape[0]
    e_ui = ui_values.shape[0]
    e_ii = ii_values.shape[0]

    ui_src = ui_edge_index[1].reshape(e_ui // _SUB, _SUB)
    ui_dst = ui_edge_index[0].reshape(e_ui // _SUB, _SUB)
    ui_val = ui_values.reshape(e_ui // _SUB, _SUB)
    ii_src = ii_edge_index[1].reshape(e_ii // _SUB, _SUB)
    ii_dst = ii_edge_index[0].reshape(e_ii // _SUB, _SUB)
    ii_val = ii_values.reshape(e_ii // _SUB, _SUB)

    zeros_ui = jnp.zeros((n_ui, _DIM), jnp.float32)
    zeros_ii = jnp.zeros((n_ii, _DIM), jnp.float32)

    x0 = jnp.concatenate([user_emb, item_emb], axis=0)
    p0, p1 = _spmm_partials(x0, ui_src, ui_dst, ui_val, zeros_ui, n_ui, e_ui, 5)
    x1 = _ew(_ADD2, p0, p1)
    p0, p1 = _spmm_partials(x1, ui_src, ui_dst, ui_val, zeros_ui, n_ui, e_ui, 5)
    x2 = _ew(_ADD2, p0, p1)
    p0, p1 = _spmm_partials(x2, ui_src, ui_dst, ui_val, zeros_ui, n_ui, e_ui, 5)
    light_out_ui = _ew(_MEAN4P, x0, x1, x2, p0, p1)
    users_emb_ui = light_out_ui[:n_users]
    items_emb_ui = light_out_ui[n_users:]

    q0, q1 = _spmm_partials(item_emb, ii_src, ii_dst, ii_val, zeros_ii,
                            n_ii, e_ii, 10)
    y1 = _ew(_ADD2, q0, q1)
    q0, q1 = _spmm_partials(y1, ii_src, ii_dst, ii_val, zeros_ii,
                            n_ii, e_ii, 10)
    items_emb_semantic, items_fused = _ew(
        _SEM_FUSE, item_emb, y1, q0, q1, items_emb_ui, n_out=2)

    return (users_emb_ui, items_emb_ui, items_emb_semantic, items_fused)


# speedup vs baseline: 31.3430x; 1.0077x over previous
import functools

import jax
import jax.numpy as jnp
from jax import lax
from jax.experimental import pallas as pl
from jax.experimental.pallas import tpu as pltpu
from jax.experimental.pallas import tpu_sc as plsc

_DIM = 16
_SUB = 125
_NCORES = 2
_NSUB = 16
_NW = _NCORES * _NSUB
_ECH = 125


def _mesh():
    return plsc.VectorSubcoreMesh(core_axis_name="c", subcore_axis_name="s")


def _params():
    return pltpu.CompilerParams(use_tc_tiling_on_sc=False,
                                needs_layout_passes=False)


@functools.lru_cache(maxsize=None)
def _make_spmm(n_rows, n_edges, crows, mode, n_a=0):
    rows_total = n_edges // _SUB
    rows_per_worker = rows_total // _NW
    chunks = rows_per_worker // crows
    assert rows_total % _NW == 0 and rows_per_worker % crows == 0
    assert chunks >= 4 and chunks % 2 == 0
    pairs = (chunks - 2) // 2
    out_rows_per_tile = n_rows // _NSUB
    assert n_rows % _NSUB == 0 and out_rows_per_tile % _ECH == 0
    nz = out_rows_per_tile // _ECH

    idx_i = pltpu.VMEM((crows, _SUB), jnp.int32)
    idx_f = pltpu.VMEM((crows, _SUB), jnp.float32)
    rows_t = pltpu.VMEM((crows, _SUB, _DIM), jnp.float32)
    ebuf = pltpu.VMEM((_ECH, _DIM), jnp.float32)
    sem = pltpu.SemaphoreType.DMA

    parts_t = jax.ShapeDtypeStruct((_NCORES, n_rows, _DIM), jnp.float32)
    x_t = jax.ShapeDtypeStruct((n_rows, _DIM), jnp.float32)
    out_type = parts_t if mode == "direct" else [x_t, parts_t]

    @functools.partial(
        pl.kernel,
        out_type=out_type,
        mesh=_mesh(),
        compiler_params=_params(),
        scratch_types=[
            idx_i, idx_i, idx_f, rows_t,
            idx_i, idx_i, idx_f, rows_t,
            pltpu.VMEM_SHARED((n_rows, _DIM), jnp.float32),
            ebuf, ebuf,
            sem, sem, sem, sem, sem, sem,
        ],
    )
    def spmm(*refs):
        if mode == "direct":
            x_hbm, src_hbm, dst_hbm, val_hbm, out_hbm = refs[:5]
            rest = refs[5:]
        elif mode == "concat":
            xa, xb, src_hbm, dst_hbm, val_hbm, x_hbm, out_hbm = refs[:7]
            rest = refs[7:]
        else:
            pin, src_hbm, dst_hbm, val_hbm, x_hbm, out_hbm = refs[:6]
            rest = refs[6:]
        (src0, dst0, val0, rows0, src1, dst1, val1, rows1, acc, ea, eb,
         isem0, gsem0, ssem0, isem1, gsem1, ssem1) = rest
        cid = lax.axis_index("c")
        sid = lax.axis_index("s")
        wid = sid * _NCORES + cid
        r0 = sid * out_rows_per_tile

        @plsc.parallel_loop(0, _ECH, unroll=5)
        def _zero(i):
            ea[i, :] = jnp.zeros((_DIM,), jnp.float32)

        for z in range(nz):
            pltpu.sync_copy(ea, acc.at[pl.ds(r0 + z * _ECH, _ECH)])

        if mode == "concat":
            arpt = n_a // _NSUB
            brpt = (n_rows - n_a) // _NSUB
            assert n_a % _NSUB == 0 and arpt % _ECH == 0 and brpt % _ECH == 0
            for z in range(arpt // _ECH):
                off = sid * arpt + z * _ECH
                pltpu.sync_copy(xa.at[pl.ds(off, _ECH)], ea)
                pltpu.sync_copy(ea, x_hbm.at[pl.ds(off, _ECH)])
            for z in range(brpt // _ECH):
                off = sid * brpt + z * _ECH
                pltpu.sync_copy(xb.at[pl.ds(off, _ECH)], ea)
                pltpu.sync_copy(ea, x_hbm.at[pl.ds(n_a + off, _ECH)])
        elif mode == "merge":
            rpt = n_rows // _NSUB
            for z in range(rpt // _ECH):
                off = sid * rpt + z * _ECH
                pltpu.sync_copy(pin.at[0].at[pl.ds(off, _ECH)], ea)
                pltpu.sync_copy(pin.at[1].at[pl.ds(off, _ECH)], eb)

                @plsc.parallel_loop(0, _ECH, unroll=5)
                def _add(i):
                    ea[i, :] = ea[i, :] + eb[i, :]

                pltpu.sync_copy(ea, x_hbm.at[pl.ds(off, _ECH)])
        plsc.subcore_barrier()

        row_base = wid * rows_per_worker
        B = [
            dict(src=src0, dst=dst0, val=val0, rows=rows0,
                 isem=isem0, gsem=gsem0, ssem=ssem0),
            dict(src=src1, dst=dst1, val=val1, rows=rows1,
                 isem=isem1, gsem=gsem1, ssem=ssem1),
        ]
        def fire_idx(k, b):
            base = row_base + k * crows
            pltpu.async_copy(src_hbm.at[pl.ds(base, crows)], b["src"], b["isem"])
            pltpu.async_copy(dst_hbm.at[pl.ds(base, crows)], b["dst"], b["isem"])
            pltpu.async_copy(val_hbm.at[pl.ds(base, crows)], b["val"], b["isem"])

        def wait_idx(b):
            for ref in (b["src"], b["dst"], b["val"]):
                pltpu.make_async_copy(
                    src_hbm.at[pl.ds(0, crows)], ref, b["isem"]).wait()

        def fire_gather(b):
            for j in range(crows):
                pltpu.async_copy(x_hbm.at[b["src"].at[j]], b["rows"].at[j],
                                 b["gsem"])

        def wait_gather(b):
            for j in range(crows):
                pltpu.make_async_copy(x_hbm.at[b["src"].at[j]],
                                      b["rows"].at[j], b["gsem"]).wait()

        def compute(b):
            val_v, rows_v = b["val"], b["rows"]
            for j in range(crows):
                @plsc.parallel_loop(0, _SUB, unroll=5)
                def _scale(i, _j=j):
                    jj = jnp.full((_DIM,), _j, jnp.int32)
                    ii = jnp.full((_DIM,), i, jnp.int32)
                    w = plsc.load_gather(val_v, [jj, ii])
                    rows_v[_j, i, :] = rows_v[_j, i, :] * w

        def fire_scatter(b):
            for j in range(crows):
                pltpu.async_copy(b["rows"].at[j], acc.at[b["dst"].at[j]],
                                 b["ssem"], add=True)

        def wait_scatter(b):
            for j in range(crows):
                pltpu.make_async_copy(b["rows"].at[j], acc.at[b["dst"].at[j]],
                                      b["ssem"]).wait()

        def body(k, b, b1, fire_next_gather=True, fire_next_idx=True):
            if fire_next_gather:
                wait_idx(b1)
                fire_gather(b1)
            wait_gather(b)
            compute(b)
            fire_scatter(b)
            wait_scatter(b)
            if fire_next_idx:
                fire_idx(k + 2, b)

        fire_idx(0, B[0])
        fire_idx(1, B[1])
        wait_idx(B[0])
        fire_gather(B[0])

        def pair_body(q, carry):
            k = 2 * q
            body(k, B[0], B[1])
            body(k + 1, B[1], B[0])
            return carry

        lax.fori_loop(0, pairs, pair_body, 0)

        body(chunks - 2, B[0], B[1], fire_next_idx=False)
        body(chunks - 1, B[1], B[0], fire_next_gather=False,
             fire_next_idx=False)

        plsc.subcore_barrier()
        pltpu.sync_copy(acc.at[pl.ds(r0, out_rows_per_tile)],
                        out_hbm.at[cid].at[pl.ds(r0, out_rows_per_tile)])

    return spmm


@functools.lru_cache(maxsize=None)
def _make_final(nu, ni):
    urpt = nu // _NSUB
    irpt = ni // _NSUB
    assert nu % _NSUB == 0 and ni % _NSUB == 0
    assert urpt % _ECH == 0 and irpt % _ECH == 0
    ebuf = pltpu.VMEM((_ECH, _DIM), jnp.float32)
    ou_t = jax.ShapeDtypeStruct((nu, _DIM), jnp.float32)
    oi_t = jax.ShapeDtypeStruct((ni, _DIM), jnp.float32)

    @functools.partial(
        pl.kernel,
        out_type=[ou_t, oi_t, oi_t],
        mesh=_mesh(),
        compiler_params=_params(),
        scratch_types=[ebuf] * 8,
    )
    def final(ue, ie, x1, x2, p3, y1, q2, ou, oi, os,
              b0, b1, b2, b3, b4, b5, b6, b7):
        sid = lax.axis_index("s")
        for z in range(urpt // _ECH):
            off = sid * urpt + z * _ECH
            pltpu.sync_copy(ue.at[pl.ds(off, _ECH)], b0)
            pltpu.sync_copy(x1.at[pl.ds(off, _ECH)], b1)
            pltpu.sync_copy(x2.at[pl.ds(off, _ECH)], b2)
            pltpu.sync_copy(p3.at[0].at[pl.ds(off, _ECH)], b3)
            pltpu.sync_copy(p3.at[1].at[pl.ds(off, _ECH)], b4)

            @plsc.parallel_loop(0, _ECH, unroll=5)
            def _users(i):
                b0[i, :] = (b0[i, :] + b1[i, :] + b2[i, :]
                            + b3[i, :] + b4[i, :]) * 0.25

            pltpu.sync_copy(b0, ou.at[pl.ds(off, _ECH)])
        for z in range(irpt // _ECH):
            t = sid * irpt + z * _ECH
            g = nu + t
            pltpu.sync_copy(ie.at[pl.ds(t, _ECH)], b0)
            pltpu.sync_copy(x1.at[pl.ds(g, _ECH)], b1)
            pltpu.sync_copy(x2.at[pl.ds(g, _ECH)], b2)
            pltpu.sync_copy(p3.at[0].at[pl.ds(g, _ECH)], b3)
            pltpu.sync_copy(p3.at[1].at[pl.ds(g, _ECH)], b4)
            pltpu.sync_copy(y1.at[pl.ds(t, _ECH)], b5)
            pltpu.sync_copy(q2.at[0].at[pl.ds(t, _ECH)], b6)
            pltpu.sync_copy(q2.at[1].at[pl.ds(t, _ECH)], b7)

            @plsc.parallel_loop(0, _ECH, unroll=5)
            def _items(i):
                li = (b0[i, :] + b1[i, :] + b2[i, :]
                      + b3[i, :] + b4[i, :]) * 0.25
                se = (b0[i, :] + b5[i, :] + b6[i, :] + b7[i, :]) * (1.0 / 3.0)
                b1[i, :] = li
                b5[i, :] = se

            pltpu.sync_copy(b1, oi.at[pl.ds(t, _ECH)])
            pltpu.sync_copy(b5, os.at[pl.ds(t, _ECH)])

    return final


_EW_BLOCK = 4096


@functools.lru_cache(maxsize=None)
def _make_ew(n_in, rows, op):
    grid = pl.cdiv(rows, _EW_BLOCK)
    spec = pl.BlockSpec((_EW_BLOCK, 128), lambda i: (i, 0))

    def body(*refs):
        refs[-1][...] = op([r[...] for r in refs[:n_in]])

    return pl.pallas_call(
        body,
        grid=(grid,),
        in_specs=[spec] * n_in,
        out_specs=spec,
        out_shape=jax.ShapeDtypeStruct((rows, 128), jnp.float32),
    )


def _ew(op, *arrays):
    rows = arrays[0].size // 128
    ins = [a.reshape(rows, 128) for a in arrays]
    return _make_ew(len(ins), rows, op)(*ins).reshape(arrays[0].shape)


_HALF2 = lambda xs: (xs[0] + xs[1]) * 0.5


def kernel(user_emb, item_emb, ui_edge_index, ui_values, ii_edge_index, ii_values):
    nu = user_emb.shape[0]
    ni = item_emb.shape[0]
    n_ui = nu + ni
    e_ui = ui_values.shape[0]
    e_ii = ii_values.shape[0]

    ui_src = ui_edge_index[1].reshape(e_ui // _SUB, _SUB)
    ui_dst = ui_edge_index[0].reshape(e_ui // _SUB, _SUB)
    ui_val = ui_values.reshape(e_ui // _SUB, _SUB)
    ii_src = ii_edge_index[1].reshape(e_ii // _SUB, _SUB)
    ii_dst = ii_edge_index[0].reshape(e_ii // _SUB, _SUB)
    ii_val = ii_values.reshape(e_ii // _SUB, _SUB)

    _, p1 = _make_spmm(n_ui, e_ui, 5, "concat", nu)(
        user_emb, item_emb, ui_src, ui_dst, ui_val)
    x1, p2 = _make_spmm(n_ui, e_ui, 5, "merge")(p1, ui_src, ui_dst, ui_val)
    x2, p3 = _make_spmm(n_ui, e_ui, 5, "merge")(p2, ui_src, ui_dst, ui_val)

    q1 = _make_spmm(ni, e_ii, 10, "direct")(item_emb, ii_src, ii_dst, ii_val)
    y1, q2 = _make_spmm(ni, e_ii, 10, "merge")(q1, ii_src, ii_dst, ii_val)

    users_emb_ui, items_emb_ui, items_emb_semantic = _make_final(nu, ni)(
        user_emb, item_emb, x1, x2, p3, y1, q2)
    items_fused = _ew(_HALF2, items_emb_ui, items_emb_semantic)

    return (users_emb_ui, items_emb_ui, items_emb_semantic, items_fused)


# speedup vs baseline: 34.5417x; 1.1021x over previous
import functools

import jax
import jax.numpy as jnp
from jax import lax
from jax.experimental import pallas as pl
from jax.experimental.pallas import tpu as pltpu
from jax.experimental.pallas import tpu_sc as plsc

_DIM = 16
_SUB = 125
_NCORES = 2
_NSUB = 16
_NW = _NCORES * _NSUB
_ECH = 125


def _mesh():
    return plsc.VectorSubcoreMesh(core_axis_name="c", subcore_axis_name="s")


def _params():
    return pltpu.CompilerParams(use_tc_tiling_on_sc=False,
                                needs_layout_passes=False)


@functools.lru_cache(maxsize=None)
def _make_spmm(n_rows, n_edges, crows, mode, n_a=0):
    rows_total = n_edges // _SUB
    rows_per_worker = rows_total // _NW
    chunks = rows_per_worker // crows
    assert rows_total % _NW == 0 and rows_per_worker % crows == 0
    assert chunks >= 4 and chunks % 2 == 0
    pairs = (chunks - 2) // 2
    out_rows_per_tile = n_rows // _NSUB
    assert n_rows % _NSUB == 0 and out_rows_per_tile % _ECH == 0
    nz = out_rows_per_tile // _ECH

    idx_i = pltpu.VMEM((crows, _SUB), jnp.int32)
    idx_f = pltpu.VMEM((crows, _SUB), jnp.float32)
    rows_t = pltpu.VMEM((crows, _SUB, _DIM), jnp.float32)
    ebuf = pltpu.VMEM((_ECH, _DIM), jnp.float32)
    sem = pltpu.SemaphoreType.DMA

    parts_t = jax.ShapeDtypeStruct((_NCORES, n_rows, _DIM), jnp.float32)
    x_t = jax.ShapeDtypeStruct((n_rows, _DIM), jnp.float32)
    out_type = parts_t if mode == "direct" else [x_t, parts_t]

    @functools.partial(
        pl.kernel,
        out_type=out_type,
        mesh=_mesh(),
        compiler_params=_params(),
        scratch_types=[
            idx_i, idx_i, idx_f, rows_t,
            idx_i, idx_i, idx_f, rows_t,
            pltpu.VMEM_SHARED((n_rows, _DIM), jnp.float32),
            ebuf, ebuf,
            sem, sem, sem, sem, sem, sem,
        ],
    )
    def spmm(*refs):
        if mode == "direct":
            x_hbm, src_hbm, dst_hbm, val_hbm, out_hbm = refs[:5]
            rest = refs[5:]
        elif mode == "concat":
            xa, xb, src_hbm, dst_hbm, val_hbm, x_hbm, out_hbm = refs[:7]
            rest = refs[7:]
        else:
            pin, src_hbm, dst_hbm, val_hbm, x_hbm, out_hbm = refs[:6]
            rest = refs[6:]
        (src0, dst0, val0, rows0, src1, dst1, val1, rows1, acc, ea, eb,
         isem0, gsem0, ssem0, isem1, gsem1, ssem1) = rest
        cid = lax.axis_index("c")
        sid = lax.axis_index("s")
        wid = sid * _NCORES + cid
        r0 = sid * out_rows_per_tile

        @plsc.parallel_loop(0, _ECH, unroll=5)
        def _zero(i):
            ea[i, :] = jnp.zeros((_DIM,), jnp.float32)

        for z in range(nz):
            pltpu.sync_copy(ea, acc.at[pl.ds(r0 + z * _ECH, _ECH)])

        if mode == "concat":
            arpt = n_a // _NSUB
            brpt = (n_rows - n_a) // _NSUB
            assert n_a % _NSUB == 0 and arpt % _ECH == 0 and brpt % _ECH == 0
            for z in range(arpt // _ECH):
                off = sid * arpt + z * _ECH
                pltpu.sync_copy(xa.at[pl.ds(off, _ECH)], ea)
                pltpu.sync_copy(ea, x_hbm.at[pl.ds(off, _ECH)])
            for z in range(brpt // _ECH):
                off = sid * brpt + z * _ECH
                pltpu.sync_copy(xb.at[pl.ds(off, _ECH)], ea)
                pltpu.sync_copy(ea, x_hbm.at[pl.ds(n_a + off, _ECH)])
        elif mode == "merge":
            rpt = n_rows // _NSUB
            for z in range(rpt // _ECH):
                off = sid * rpt + z * _ECH
                pltpu.sync_copy(pin.at[0].at[pl.ds(off, _ECH)], ea)
                pltpu.sync_copy(pin.at[1].at[pl.ds(off, _ECH)], eb)

                @plsc.parallel_loop(0, _ECH, unroll=5)
                def _add(i):
                    ea[i, :] = ea[i, :] + eb[i, :]

                pltpu.sync_copy(ea, x_hbm.at[pl.ds(off, _ECH)])
        plsc.subcore_barrier()

        row_base = wid * rows_per_worker
        B = [
            dict(src=src0, dst=dst0, val=val0, rows=rows0,
                 isem=isem0, gsem=gsem0, ssem=ssem0),
            dict(src=src1, dst=dst1, val=val1, rows=rows1,
                 isem=isem1, gsem=gsem1, ssem=ssem1),
        ]
        def fire_idx(k, b):
            base = row_base + k * crows
            pltpu.async_copy(src_hbm.at[pl.ds(base, crows)], b["src"], b["isem"])
            pltpu.async_copy(dst_hbm.at[pl.ds(base, crows)], b["dst"], b["isem"])
            pltpu.async_copy(val_hbm.at[pl.ds(base, crows)], b["val"], b["isem"])

        def wait_idx(b):
            for ref in (b["src"], b["dst"], b["val"]):
                pltpu.make_async_copy(
                    src_hbm.at[pl.ds(0, crows)], ref, b["isem"]).wait()

        def fire_gather(b):
            for j in range(crows):
                pltpu.async_copy(x_hbm.at[b["src"].at[j]], b["rows"].at[j],
                                 b["gsem"])

        def wait_gather(b):
            for j in range(crows):
                pltpu.make_async_copy(x_hbm.at[b["src"].at[j]],
                                      b["rows"].at[j], b["gsem"]).wait()

        def compute(b):
            val_v, rows_v = b["val"], b["rows"]
            for j in range(crows):
                @plsc.parallel_loop(0, _SUB, unroll=5)
                def _scale(i, _j=j):
                    jj = jnp.full((_DIM,), _j, jnp.int32)
                    ii = jnp.full((_DIM,), i, jnp.int32)
                    w = plsc.load_gather(val_v, [jj, ii])
                    rows_v[_j, i, :] = rows_v[_j, i, :] * w

        def fire_scatter(b):
            for j in range(crows):
                pltpu.async_copy(b["rows"].at[j], acc.at[b["dst"].at[j]],
                                 b["ssem"], add=True)

        def wait_scatter(b):
            for j in range(crows):
                pltpu.make_async_copy(b["rows"].at[j], acc.at[b["dst"].at[j]],
                                      b["ssem"]).wait()

        def body(k, b, b1, fire_next_gather=True, fire_next_idx=True):
            if fire_next_gather:
                wait_idx(b1)
                fire_gather(b1)
            wait_gather(b)
            compute(b)
            fire_scatter(b)
            wait_scatter(b)
            if fire_next_idx:
                fire_idx(k + 2, b)

        fire_idx(0, B[0])
        fire_idx(1, B[1])
        wait_idx(B[0])
        fire_gather(B[0])

        def pair_body(q, carry):
            k = 2 * q
            body(k, B[0], B[1])
            body(k + 1, B[1], B[0])
            return carry

        lax.fori_loop(0, pairs, pair_body, 0)

        body(chunks - 2, B[0], B[1], fire_next_idx=False)
        body(chunks - 1, B[1], B[0], fire_next_gather=False,
             fire_next_idx=False)

        plsc.subcore_barrier()
        pltpu.sync_copy(acc.at[pl.ds(r0, out_rows_per_tile)],
                        out_hbm.at[cid].at[pl.ds(r0, out_rows_per_tile)])

    return spmm


@functools.lru_cache(maxsize=None)
def _make_final(nu, ni):
    fch = 625
    urpt = nu // _NSUB
    irpt = ni // _NSUB
    assert nu % _NSUB == 0 and ni % _NSUB == 0
    assert urpt % fch == 0 and irpt % fch == 0
    ebuf = pltpu.VMEM((fch, _DIM), jnp.float32)
    sem = pltpu.SemaphoreType.DMA
    ou_t = jax.ShapeDtypeStruct((nu, _DIM), jnp.float32)
    oi_t = jax.ShapeDtypeStruct((ni, _DIM), jnp.float32)

    @functools.partial(
        pl.kernel,
        out_type=[ou_t, oi_t, oi_t],
        mesh=_mesh(),
        compiler_params=_params(),
        scratch_types=[ebuf] * 8 + [sem],
    )
    def final(ue, ie, x1, x2, p3, y1, q2, ou, oi, os,
              b0, b1, b2, b3, b4, b5, b6, b7, dsem):
        sid = lax.axis_index("s")

        def load(pairs):
            for src, buf in pairs:
                pltpu.async_copy(src, buf, dsem)
            for src, buf in pairs:
                pltpu.make_async_copy(src, buf, dsem).wait()

        for z in range(urpt // fch):
            off = sid * urpt + z * fch
            load([(ue.at[pl.ds(off, fch)], b0),
                  (x1.at[pl.ds(off, fch)], b1),
                  (x2.at[pl.ds(off, fch)], b2),
                  (p3.at[0].at[pl.ds(off, fch)], b3),
                  (p3.at[1].at[pl.ds(off, fch)], b4)])

            @plsc.parallel_loop(0, fch, unroll=5)
            def _users(i):
                b0[i, :] = (b0[i, :] + b1[i, :] + b2[i, :]
                            + b3[i, :] + b4[i, :]) * 0.25

            pltpu.sync_copy(b0, ou.at[pl.ds(off, fch)])
        for z in range(irpt // fch):
            t = sid * irpt + z * fch
            g = nu + t
            load([(ie.at[pl.ds(t, fch)], b0),
                  (x1.at[pl.ds(g, fch)], b1),
                  (x2.at[pl.ds(g, fch)], b2),
                  (p3.at[0].at[pl.ds(g, fch)], b3),
                  (p3.at[1].at[pl.ds(g, fch)], b4),
                  (y1.at[pl.ds(t, fch)], b5),
                  (q2.at[0].at[pl.ds(t, fch)], b6),
                  (q2.at[1].at[pl.ds(t, fch)], b7)])

            @plsc.parallel_loop(0, fch, unroll=5)
            def _items(i):
                li = (b0[i, :] + b1[i, :] + b2[i, :]
                      + b3[i, :] + b4[i, :]) * 0.25
                se = (b0[i, :] + b5[i, :] + b6[i, :] + b7[i, :]) * (1.0 / 3.0)
                b1[i, :] = li
                b5[i, :] = se

            pltpu.async_copy(b1, oi.at[pl.ds(t, fch)], dsem)
            pltpu.async_copy(b5, os.at[pl.ds(t, fch)], dsem)
            pltpu.make_async_copy(b1, oi.at[pl.ds(t, fch)], dsem).wait()
            pltpu.make_async_copy(b5, os.at[pl.ds(t, fch)], dsem).wait()

    return final


_EW_BLOCK = 4096


@functools.lru_cache(maxsize=None)
def _make_ew(n_in, rows, op):
    grid = pl.cdiv(rows, _EW_BLOCK)
    spec = pl.BlockSpec((_EW_BLOCK, 128), lambda i: (i, 0))

    def body(*refs):
        refs[-1][...] = op([r[...] for r in refs[:n_in]])

    return pl.pallas_call(
        body,
        grid=(grid,),
        in_specs=[spec] * n_in,
        out_specs=spec,
        out_shape=jax.ShapeDtypeStruct((rows, 128), jnp.float32),
    )


def _ew(op, *arrays):
    rows = arrays[0].size // 128
    ins = [a.reshape(rows, 128) for a in arrays]
    return _make_ew(len(ins), rows, op)(*ins).reshape(arrays[0].shape)


_HALF2 = lambda xs: (xs[0] + xs[1]) * 0.5


def kernel(user_emb, item_emb, ui_edge_index, ui_values, ii_edge_index, ii_values):
    nu = user_emb.shape[0]
    ni = item_emb.shape[0]
    n_ui = nu + ni
    e_ui = ui_values.shape[0]
    e_ii = ii_values.shape[0]

    ui_src = ui_edge_index[1].reshape(e_ui // _SUB, _SUB)
    ui_dst = ui_edge_index[0].reshape(e_ui // _SUB, _SUB)
    ui_val = ui_values.reshape(e_ui // _SUB, _SUB)
    ii_src = ii_edge_index[1].reshape(e_ii // _SUB, _SUB)
    ii_dst = ii_edge_index[0].reshape(e_ii // _SUB, _SUB)
    ii_val = ii_values.reshape(e_ii // _SUB, _SUB)

    q1 = _make_spmm(ni, e_ii, 10, "direct")(item_emb, ii_src, ii_dst, ii_val)

    _, p1 = _make_spmm(n_ui, e_ui, 5, "concat", nu)(
        user_emb, item_emb, ui_src, ui_dst, ui_val)
    x1, p2 = _make_spmm(n_ui, e_ui, 5, "merge")(p1, ui_src, ui_dst, ui_val)
    x2, p3 = _make_spmm(n_ui, e_ui, 5, "merge")(p2, ui_src, ui_dst, ui_val)

    y1, q2 = _make_spmm(ni, e_ii, 10, "merge")(q1, ii_src, ii_dst, ii_val)

    users_emb_ui, items_emb_ui, items_emb_semantic = _make_final(nu, ni)(
        user_emb, item_emb, x1, x2, p3, y1, q2)
    items_fused = _ew(_HALF2, items_emb_ui, items_emb_semantic)

    return (users_emb_ui, items_emb_ui, items_emb_semantic, items_fused)
